# Initial kernel scaffold; baseline (speedup 1.0000x reference)
import jax
import jax.numpy as jnp
from jax.experimental import pallas as pl


def kernel(x, edge_index, edge, A_values, W0, b0, W1, b1, W2, b2, fe_W1, fe_b1, fe_W2, fe_b2, fn_W1, fn_b1, fn_W2, fn_b2, gp_W1, gp_b1, gp_W2, gp_b2, alpha):
    raise NotImplementedError("write your pallas kernel here")


# speedup vs baseline: 2.0784x; 2.0784x over previous
import functools

import jax
import jax.numpy as jnp
from jax import lax
from jax.experimental import pallas as pl
from jax.experimental.pallas import tpu as pltpu
from jax.experimental.pallas import tpu_sc as plsc

N = 10000
NP = 10240
E = 160000
EP = 163840
B = 1024
D_IN = 128
HID = 128
OUT = 64

NC = 2
NS = 16
NW = NC * NS
ET = EP // NW
ETR = ET // 128
NROWS = NP // NS
CS = 128
NCHUNK = (2 * B) // CS
ROWW = NP
DUMMY = CS * ROWW

_mesh = plsc.VectorSubcoreMesh(core_axis_name="c", subcore_axis_name="s")


def _iota16():
    return lax.iota(jnp.int32, 16)


def _sc_stats_body(col3, ew3, ones3, nodes4, degp, nsfp, fslot_hbm,
                   col_v, ew_v, ones_v, nd_v, sv_v, zf_v, mi_v,
                   dacc, nacc, fsl, sem):
    c = lax.axis_index("c")
    s = lax.axis_index("s")
    w = s * NC + c

    for i in range(NROWS // 16):
        zf_v[pl.ds(i * 16, 16)] = jnp.zeros((16,), jnp.float32)
        mi_v[pl.ds(i * 16, 16)] = jnp.full((16,), -1, jnp.int32)
    pltpu.sync_copy(zf_v, dacc.at[pl.ds(s * NROWS, NROWS)])
    pltpu.sync_copy(zf_v, nacc.at[pl.ds(s * NROWS, NROWS)])
    pltpu.sync_copy(mi_v, fsl.at[pl.ds(s * NROWS, NROWS)])
    plsc.subcore_barrier()

    pltpu.sync_copy(col3.at[w], col_v)
    pltpu.sync_copy(ew3.at[w], ew_v)
    pltpu.sync_copy(ones3.at[w], ones_v)

    descs = []
    for j in range(ETR):
        descs.append(pltpu.async_copy(ones_v.at[j], dacc.at[col_v.at[j]],
                                      sem, add=True))
        descs.append(pltpu.async_copy(ew_v.at[j], nacc.at[col_v.at[j]],
                                      sem, add=True))
    for d in descs:
        d.wait()

    @pl.when(c == 0)
    def _():
        pltpu.sync_copy(nodes4.at[s], nd_v)
        for k in range(8):
            sv_v[0, pl.ds(k * 16, 16)] = s * 128 + k * 16 + _iota16()
        pltpu.sync_copy(sv_v.at[0], fsl.at[nd_v.at[0]])

    plsc.subcore_barrier()

    pltpu.sync_copy(dacc.at[pl.ds(s * NROWS, NROWS)],
                    degp.at[c, pl.ds(s * NROWS, NROWS)])
    pltpu.sync_copy(nacc.at[pl.ds(s * NROWS, NROWS)],
                    nsfp.at[c, pl.ds(s * NROWS, NROWS)])

    @pl.when(c == 0)
    def _():
        pltpu.sync_copy(fsl.at[pl.ds(s * NROWS, NROWS)],
                        fslot_hbm.at[pl.ds(s * NROWS, NROWS)])


_sc_stats = pl.kernel(
    _sc_stats_body,
    out_type=(
        jax.ShapeDtypeStruct((NC, NP), jnp.float32),
        jax.ShapeDtypeStruct((NC, NP), jnp.float32),
        jax.ShapeDtypeStruct((NP,), jnp.int32),
    ),
    mesh=_mesh,
    compiler_params=pltpu.CompilerParams(needs_layout_passes=False),
    scratch_types=(
        pltpu.VMEM((ETR, 128), jnp.int32),
        pltpu.VMEM((ETR, 128), jnp.float32),
        pltpu.VMEM((ETR, 128), jnp.float32),
        pltpu.VMEM((1, 128), jnp.int32),
        pltpu.VMEM((1, 128), jnp.int32),
        pltpu.VMEM((NROWS,), jnp.float32),
        pltpu.VMEM((NROWS,), jnp.int32),
        pltpu.VMEM_SHARED((NP,), jnp.float32),
        pltpu.VMEM_SHARED((NP,), jnp.float32),
        pltpu.VMEM_SHARED((NP,), jnp.int32),
        pltpu.SemaphoreType.DMA,
    ),
)


def _sc_reps_body(fslot_hbm, row3, e0_hbm, e1_hbm, rep3, repu, repv,
                  fsl_v, row_v, rep_v, q_v, qr_v):
    c = lax.axis_index("c")
    s = lax.axis_index("s")
    w = s * NC + c

    pltpu.sync_copy(fslot_hbm, fsl_v)
    pltpu.sync_copy(row3.at[w], row_v)

    for j in range(ETR):
        for k in range(8):
            idx = row_v[j, pl.ds(k * 16, 16)]
            rep_v[j, pl.ds(k * 16, 16)] = plsc.load_gather(fsl_v, [idx])
    pltpu.sync_copy(rep_v, rep3.at[w])

    qn = B // NW
    pltpu.sync_copy(e0_hbm.at[pl.ds(w * qn, qn)], q_v)
    for k in range(qn // 16):
        qr_v[pl.ds(k * 16, 16)] = plsc.load_gather(
            fsl_v, [q_v[pl.ds(k * 16, 16)]])
    pltpu.sync_copy(qr_v, repu.at[pl.ds(w * qn, qn)])

    pltpu.sync_copy(e1_hbm.at[pl.ds(w * qn, qn)], q_v)
    for k in range(qn // 16):
        qr_v[pl.ds(k * 16, 16)] = plsc.load_gather(
            fsl_v, [q_v[pl.ds(k * 16, 16)]])
    pltpu.sync_copy(qr_v, repv.at[pl.ds(w * qn, qn)])


_sc_reps = pl.kernel(
    _sc_reps_body,
    out_type=(
        jax.ShapeDtypeStruct((NW, ETR, 128), jnp.int32),
        jax.ShapeDtypeStruct((B,), jnp.int32),
        jax.ShapeDtypeStruct((B,), jnp.int32),
    ),
    mesh=_mesh,
    compiler_params=pltpu.CompilerParams(needs_layout_passes=False),
    scratch_types=(
        pltpu.VMEM((NP,), jnp.int32),
        pltpu.VMEM((ETR, 128), jnp.int32),
        pltpu.VMEM((ETR, 128), jnp.int32),
        pltpu.VMEM((32,), jnp.int32),
        pltpu.VMEM((32,), jnp.int32),
    ),
)


def _sc_prop_body(h, row3, col3, zs, pp, row_v, col_v, rb0, rb1, zb_v,
                  acc, g0, g1, s0, s1):
    c = lax.axis_index("c")
    s = lax.axis_index("s")
    w = s * NC + c
    sh = 7 if h == 128 else 6
    nz = 40

    for r in range(nz):
        for k in range(h // 16):
            zb_v[r, pl.ds(k * 16, 16)] = jnp.zeros((16,), jnp.float32)
    for t in range(NROWS // nz):
        pltpu.sync_copy(zb_v, acc.at[pl.ds(s * NROWS + t * nz, nz)])
    plsc.subcore_barrier()

    pltpu.sync_copy(row3.at[w], row_v)
    pltpu.sync_copy(col3.at[w], col_v)

    rbufs = (rb0, rb1)
    gsems = (g0, g1)
    ssems = (s0, s1)
    gd = [None] * ETR
    sd = [None] * ETR
    gd[0] = pltpu.async_copy(zs.at[col_v.at[0]], rbufs[0], gsems[0])
    for j in range(ETR):
        gd[j].wait()
        if j + 1 < ETR:
            if j >= 1:
                sd[j - 1].wait()
            gd[j + 1] = pltpu.async_copy(zs.at[col_v.at[j + 1]],
                                         rbufs[(j + 1) % 2],
                                         gsems[(j + 1) % 2])
        sd[j] = pltpu.async_copy(rbufs[j % 2], acc.at[row_v.at[j]],
                                 ssems[j % 2], add=True)
    sd[ETR - 2].wait()
    sd[ETR - 1].wait()
    plsc.subcore_barrier()

    pltpu.sync_copy(acc.at[pl.ds(s * NROWS, NROWS)],
                    pp.at[c, pl.ds(s * NROWS, NROWS)])


def _make_sc_prop(h):
    return pl.kernel(
        functools.partial(_sc_prop_body, h),
        out_type=jax.ShapeDtypeStruct((NC, NP, h), jnp.float32),
        mesh=_mesh,
        compiler_params=pltpu.CompilerParams(needs_layout_passes=False),
        scratch_types=(
            pltpu.VMEM((ETR, 128), jnp.int32),
            pltpu.VMEM((ETR, 128), jnp.int32),
            pltpu.VMEM((128, h), jnp.float32),
            pltpu.VMEM((128, h), jnp.float32),
            pltpu.VMEM((40, h), jnp.float32),
            pltpu.VMEM_SHARED((NP, h), jnp.float32),
            pltpu.SemaphoreType.DMA,
            pltpu.SemaphoreType.DMA,
            pltpu.SemaphoreType.DMA,
            pltpu.SemaphoreType.DMA,
        ),
    )


_sc_prop_hid = _make_sc_prop(HID)


def _sc_rows_body(rep2, col2, val2, rows_hbm,
                  rep_v, col_v, val_v, fi_v, zb_v, buf, sem):
    c = lax.axis_index("c")
    s = lax.axis_index("s")
    nrow2 = ETR * 2

    pltpu.sync_copy(rep2.at[s], rep_v)
    pltpu.sync_copy(col2.at[s], col_v)
    pltpu.sync_copy(val2.at[s], val_v)

    for i in range(4096 // 16):
        zb_v[pl.ds(i * 16, 16)] = jnp.zeros((16,), jnp.float32)

    words_per_tile = (CS * ROWW) // NS
    nzcopy = words_per_tile // 4096

    def chunk_body(chunk, _):
        cc = c * (NCHUNK // NC) + chunk
        base = cc * CS

        zd = []
        for z in range(nzcopy):
            zd.append(pltpu.async_copy(
                zb_v, buf.at[pl.ds(s * words_per_tile + z * 4096, 4096)],
                sem))
        for d in zd:
            d.wait()

        @pl.when(s == 0)
        def _():
            pltpu.sync_copy(zb_v.at[pl.ds(0, 8)],
                            buf.at[pl.ds(CS * ROWW, 8)])
        plsc.subcore_barrier()

        for j in range(nrow2):
            for k in range(8):
                rep = rep_v[j, pl.ds(k * 16, 16)]
                col = col_v[j, pl.ds(k * 16, 16)]
                inchunk = (rep >= base) & (rep < base + CS)
                fi_v[j, pl.ds(k * 16, 16)] = jnp.where(
                    inchunk, (rep - base) * ROWW + col, DUMMY)

        descs = []
        for j in range(nrow2):
            descs.append(pltpu.async_copy(val_v.at[j], buf.at[fi_v.at[j]],
                                          sem, add=True))
        for d in descs:
            d.wait()
        plsc.subcore_barrier()

        dd = []
        for r in range(CS // NS):
            slot = s * (CS // NS) + r
            dd.append(pltpu.async_copy(
                buf.at[pl.ds(slot * ROWW, ROWW)],
                rows_hbm.at[base + slot], sem))
        for d in dd:
            d.wait()
        plsc.subcore_barrier()
        return 0

    lax.fori_loop(0, NCHUNK // NC, chunk_body, 0)


_sc_rows = pl.kernel(
    _sc_rows_body,
    out_type=jax.ShapeDtypeStruct((2 * B, ROWW), jnp.float32),
    mesh=_mesh,
    compiler_params=pltpu.CompilerParams(needs_layout_passes=False),
    scratch_types=(
        pltpu.VMEM((ETR * 2, 128), jnp.int32),
        pltpu.VMEM((ETR * 2, 128), jnp.int32),
        pltpu.VMEM((ETR * 2, 128), jnp.float32),
        pltpu.VMEM((ETR * 2, 128), jnp.int32),
        pltpu.VMEM((4096,), jnp.float32),
        pltpu.VMEM_SHARED((CS * ROWW + 8,), jnp.float32),
        pltpu.SemaphoreType.DMA,
    ),
)


def _fe_body(a_ref, w1, b1, w2, b2, o_ref):
    x = a_ref[...]
    acc = jnp.full(x.shape, b2[0], jnp.float32)
    for k in range(8):
        acc = acc + jax.nn.relu(x * w1[0, k] + b1[k]) * w2[k, 0]
    o_ref[...] = acc


def _fe_mlp(a, w1, b1, w2, b2):
    m = E // 128
    return pl.pallas_call(
        _fe_body,
        in_specs=[
            pl.BlockSpec((m, 128), lambda: (0, 0)),
            pl.BlockSpec(memory_space=pltpu.SMEM),
            pl.BlockSpec(memory_space=pltpu.SMEM),
            pl.BlockSpec(memory_space=pltpu.SMEM),
            pl.BlockSpec(memory_space=pltpu.SMEM),
        ],
        out_specs=pl.BlockSpec((m, 128), lambda: (0, 0)),
        out_shape=jax.ShapeDtypeStruct((m, 128), jnp.float32),
    )(a, w1, b1, w2, b2)


def _node_prep_body(degp, nsfp, w1, b1, w2, b2, dinv_ref, g_ref):
    deg = 1.0 + degp[0, :] + degp[1, :]
    dinv = lax.rsqrt(jnp.maximum(deg, 1e-12))
    dinv_ref[...] = dinv[:, None]
    nsf = (nsfp[0, :] + nsfp[1, :])[:, None]
    hid = jax.nn.relu(nsf * w1[...] + b1[...][None, :])
    w2r = w2[...].reshape(1, HID)
    fn = jnp.sum(hid * w2r, axis=1, keepdims=True) + b2[...][None, :]
    g_ref[...] = fn * fn


def _node_prep(degp, nsfp, w1, b1, w2, b2):
    return pl.pallas_call(
        _node_prep_body,
        out_shape=(
            jax.ShapeDtypeStruct((NP, 1), jnp.float32),
            jax.ShapeDtypeStruct((NP, 1), jnp.float32),
        ),
    )(degp, nsfp, w1, b1, w2, b2)


def _ms_body(h_ref, w_ref, dinv_ref, o_ref):
    z = jnp.dot(h_ref[...], w_ref[...], preferred_element_type=jnp.float32)
    o_ref[...] = z * dinv_ref[...]


def _matmul_scale(h, w, dinv):
    din, dout = w.shape
    bm = 1024
    return pl.pallas_call(
        _ms_body,
        grid=(NP // bm,),
        in_specs=[
            pl.BlockSpec((bm, din), lambda i: (i, 0)),
            pl.BlockSpec((din, dout), lambda i: (0, 0)),
            pl.BlockSpec((bm, 1), lambda i: (i, 0)),
        ],
        out_specs=pl.BlockSpec((bm, dout), lambda i: (i, 0)),
        out_shape=jax.ShapeDtypeStruct((NP, dout), jnp.float32),
    )(h, w, dinv)


def _ep_body(relu, pp_ref, zs_ref, dinv_ref, b_ref, o_ref):
    v = (pp_ref[0] + pp_ref[1] + zs_ref[...]) * dinv_ref[...] + b_ref[...]
    o_ref[...] = jax.nn.relu(v) if relu else v


def _epilogue(pp, zs, dinv, b, relu):
    h = zs.shape[1]
    bm = 1024
    return pl.pallas_call(
        functools.partial(_ep_body, relu),
        grid=(NP // bm,),
        in_specs=[
            pl.BlockSpec((NC, bm, h), lambda i: (0, i, 0)),
            pl.BlockSpec((bm, h), lambda i: (i, 0)),
            pl.BlockSpec((bm, 1), lambda i: (i, 0)),
            pl.BlockSpec((1, h), lambda i: (0, 0)),
        ],
        out_specs=pl.BlockSpec((bm, h), lambda i: (i, 0)),
        out_shape=jax.ShapeDtypeStruct((NP, h), jnp.float32),
    )(pp, zs, dinv, b)


def _pair_body(ru, rv, e0, e1, rowsu_ref, rowsv_ref, h3u_ref, h3v_ref,
               g_ref, osr_ref, of_ref):
    prod = rowsu_ref[0] * rowsv_ref[0] * g_ref[0]
    osr_ref[...] = jnp.full((1, 1, 128), jnp.sum(prod), jnp.float32)
    of_ref[...] = jnp.full((1, 1, 128),
                           jnp.sum(h3u_ref[...] * h3v_ref[...]), jnp.float32)


def _pair(rows3, h3, g3, ru, rv, e0, e1):
    grid_spec = pltpu.PrefetchScalarGridSpec(
        num_scalar_prefetch=4,
        grid=(B,),
        in_specs=[
            pl.BlockSpec((1, NP // 128, 128),
                         lambda b, ru, rv, e0, e1: (ru[b], 0, 0)),
            pl.BlockSpec((1, NP // 128, 128),
                         lambda b, ru, rv, e0, e1: (rv[b], 0, 0)),
            pl.BlockSpec((1, 1, 128), lambda b, ru, rv, e0, e1: (e0[b], 0, 0)),
            pl.BlockSpec((1, 1, 128), lambda b, ru, rv, e0, e1: (e1[b], 0, 0)),
            pl.BlockSpec((1, NP // 128, 128),
                         lambda b, ru, rv, e0, e1: (0, 0, 0)),
        ],
        out_specs=[
            pl.BlockSpec((1, 1, 128), lambda b, ru, rv, e0, e1: (b, 0, 0)),
            pl.BlockSpec((1, 1, 128), lambda b, ru, rv, e0, e1: (b, 0, 0)),
        ],
    )
    return pl.pallas_call(
        _pair_body,
        grid_spec=grid_spec,
        out_shape=(
            jax.ShapeDtypeStruct((B, 1, 128), jnp.float32),
            jax.ShapeDtypeStruct((B, 1, 128), jnp.float32),
        ),
    )(ru, rv, e0, e1, rows3, rows3, h3, h3, g3)


def _final_body(osr_ref, of_ref, w1, b1, w2, b2, alpha_ref,
                out_ref, os_ref):
    x = osr_ref[...]
    hid = jax.nn.relu(x * w1[...] + b1[...][None, :])
    w2r = w2[...].reshape(1, HID)
    t = jnp.sum(hid * w2r, axis=1, keepdims=True) + b2[...][None, :]
    os_v = 1.0 / (1.0 + jnp.exp(-t))
    a2 = alpha_ref[...]
    m = jnp.max(a2)
    e = jnp.exp(a2 - m)
    ssum = jnp.sum(e)
    ii = lax.broadcasted_iota(jnp.int32, (1, 2), 1)
    a0 = jnp.sum(jnp.where(ii == 0, e, 0.0)) / ssum
    a1 = jnp.sum(jnp.where(ii == 1, e, 0.0)) / ssum
    os_ref[...] = os_v
    out_ref[...] = a0 * os_v + a1 * of_ref[...] + 1e-15


def _final(osr, of, w1, b1, w2, b2, alpha):
    return pl.pallas_call(
        _final_body,
        out_shape=(
            jax.ShapeDtypeStruct((B, 1), jnp.float32),
            jax.ShapeDtypeStruct((B, 1), jnp.float32),
        ),
    )(osr, of, w1, b1, w2, b2, alpha)


def kernel(x, edge_index, edge, A_values, W0, b0, W1, b1, W2, b2,
           fe_W1, fe_b1, fe_W2, fe_b2, fn_W1, fn_b1, fn_W2, fn_b2,
           gp_W1, gp_b1, gp_W2, gp_b2, alpha):
    row, col = edge_index[0], edge_index[1]
    e0, e1 = edge[0], edge[1]

    pad = EP - E
    row_p = jnp.pad(row, (0, pad), constant_values=NP - 1)
    col_p = jnp.pad(col, (0, pad), constant_values=NP - 1)
    row3 = row_p.reshape(NW, ETR, 128)
    col3 = col_p.reshape(NW, ETR, 128)
    ones3 = jnp.pad(jnp.ones((E,), jnp.float32), (0, pad)
                    ).reshape(NW, ETR, 128)
    nodes4 = jnp.concatenate([e0, e1]).reshape(NS, 1, 128)

    ew = _fe_mlp(A_values.reshape(E // 128, 128),
                 fe_W1, fe_b1, fe_W2, fe_b2).reshape(E)
    ew3 = jnp.pad(ew, (0, pad)).reshape(NW, ETR, 128)

    degp, nsfp, fslot = _sc_stats(col3, ew3, ones3, nodes4)
    dinv, g = _node_prep(degp, nsfp, fn_W1, fn_b1, fn_W2, fn_b2)

    rep3, repu, repv = _sc_reps(fslot, row3, e0, e1)

    x_p = jnp.pad(x, ((0, NP - N), (0, 0)))
    zs0 = _matmul_scale(x_p, W0, dinv)
    pp0 = _sc_prop_hid(row3, col3, zs0)
    h1 = _epilogue(pp0, zs0, dinv, b0[None, :], True)
    zs1 = _matmul_scale(h1, W1, dinv)
    pp1 = _sc_prop_hid(row3, col3, zs1)
    h2 = _epilogue(pp1, zs1, dinv, b1[None, :], True)
    W2p = jnp.pad(W2, ((0, 0), (0, 128 - OUT)))
    b2p = jnp.pad(b2, (0, 128 - OUT))
    zs2 = _matmul_scale(h2, W2p, dinv)
    pp2 = _sc_prop_hid(row3, col3, zs2)
    h3 = _epilogue(pp2, zs2, dinv, b2p[None, :], False)

    val2 = jnp.pad(A_values, (0, pad)).reshape(NS, ETR * 2, 128)
    rep2 = rep3.reshape(NS, ETR * 2, 128)
    col2 = col_p.reshape(NS, ETR * 2, 128)
    rows = _sc_rows(rep2, col2, val2)

    rows3 = rows.reshape(2 * B, NP // 128, 128)
    g3 = g.reshape(1, NP // 128, 128)
    h3r = h3.reshape(NP, 1, 128)
    osr_f, of_f = _pair(rows3, h3r, g3, repu, repv, e0, e1)
    osr = osr_f[:, 0, :1]
    of = of_f[:, 0, :1]

    out, out_struct = _final(osr, of, gp_W1, gp_b1, gp_W2, gp_b2, alpha)
    return out, out_struct, of


# speedup vs baseline: 3.7794x; 1.8184x over previous
import functools

import jax
import jax.numpy as jnp
from jax import lax
from jax.experimental import pallas as pl
from jax.experimental.pallas import tpu as pltpu
from jax.experimental.pallas import tpu_sc as plsc

N = 10000
NP = 10240
E = 160000
EP = 163840
B = 1024
D_IN = 128
HID = 128
OUT = 64

NC = 2
NS = 16
NW = NC * NS
ET = EP // NW
ETR = ET // 128
NROWS = NP // NS
CS = 128
NCHUNK = (2 * B) // CS
ROWW = NP
DUMMY = CS * ROWW

_mesh = plsc.VectorSubcoreMesh(core_axis_name="c", subcore_axis_name="s")


def _iota16():
    return lax.iota(jnp.int32, 16)


def _sc_stats_body(col3, ew3, ones3, nodes4, degp, nsfp, fslot_hbm,
                   col_v, ew_v, ones_v, nd_v, sv_v, zf_v, mi_v,
                   dacc, nacc, fsl, sem):
    c = lax.axis_index("c")
    s = lax.axis_index("s")
    w = s * NC + c

    for i in range(NROWS // 16):
        zf_v[pl.ds(i * 16, 16)] = jnp.zeros((16,), jnp.float32)
        mi_v[pl.ds(i * 16, 16)] = jnp.full((16,), -1, jnp.int32)
    pltpu.sync_copy(zf_v, dacc.at[pl.ds(s * NROWS, NROWS)])
    pltpu.sync_copy(zf_v, nacc.at[pl.ds(s * NROWS, NROWS)])
    pltpu.sync_copy(mi_v, fsl.at[pl.ds(s * NROWS, NROWS)])
    plsc.subcore_barrier()

    pltpu.sync_copy(col3.at[w], col_v)
    pltpu.sync_copy(ew3.at[w], ew_v)
    pltpu.sync_copy(ones3.at[w], ones_v)

    descs = []
    for j in range(ETR):
        descs.append(pltpu.async_copy(ones_v.at[j], dacc.at[col_v.at[j]],
                                      sem, add=True))
        descs.append(pltpu.async_copy(ew_v.at[j], nacc.at[col_v.at[j]],
                                      sem, add=True))
    for d in descs:
        d.wait()

    @pl.when(c == 0)
    def _():
        pltpu.sync_copy(nodes4.at[s], nd_v)
        for k in range(8):
            sv_v[0, pl.ds(k * 16, 16)] = s * 128 + k * 16 + _iota16()
        pltpu.sync_copy(sv_v.at[0], fsl.at[nd_v.at[0]])

    plsc.subcore_barrier()

    pltpu.sync_copy(dacc.at[pl.ds(s * NROWS, NROWS)],
                    degp.at[c, pl.ds(s * NROWS, NROWS)])
    pltpu.sync_copy(nacc.at[pl.ds(s * NROWS, NROWS)],
                    nsfp.at[c, pl.ds(s * NROWS, NROWS)])

    @pl.when(c == 0)
    def _():
        pltpu.sync_copy(fsl.at[pl.ds(s * NROWS, NROWS)],
                        fslot_hbm.at[pl.ds(s * NROWS, NROWS)])


_sc_stats = pl.kernel(
    _sc_stats_body,
    out_type=(
        jax.ShapeDtypeStruct((NC, NP), jnp.float32),
        jax.ShapeDtypeStruct((NC, NP), jnp.float32),
        jax.ShapeDtypeStruct((NP,), jnp.int32),
    ),
    mesh=_mesh,
    compiler_params=pltpu.CompilerParams(needs_layout_passes=False),
    scratch_types=(
        pltpu.VMEM((ETR, 128), jnp.int32),
        pltpu.VMEM((ETR, 128), jnp.float32),
        pltpu.VMEM((ETR, 128), jnp.float32),
        pltpu.VMEM((1, 128), jnp.int32),
        pltpu.VMEM((1, 128), jnp.int32),
        pltpu.VMEM((NROWS,), jnp.float32),
        pltpu.VMEM((NROWS,), jnp.int32),
        pltpu.VMEM_SHARED((NP,), jnp.float32),
        pltpu.VMEM_SHARED((NP,), jnp.float32),
        pltpu.VMEM_SHARED((NP,), jnp.int32),
        pltpu.SemaphoreType.DMA,
    ),
)


def _sc_reps_body(fslot_hbm, row3, e0_hbm, e1_hbm, rep3, repu, repv,
                  fsl_v, row_v, rep_v, q_v, qr_v):
    c = lax.axis_index("c")
    s = lax.axis_index("s")
    w = s * NC + c

    pltpu.sync_copy(fslot_hbm, fsl_v)
    pltpu.sync_copy(row3.at[w], row_v)

    for j in range(ETR):
        for k in range(8):
            idx = row_v[j, pl.ds(k * 16, 16)]
            rep_v[j, pl.ds(k * 16, 16)] = plsc.load_gather(fsl_v, [idx])
    pltpu.sync_copy(rep_v, rep3.at[w])

    qn = B // NW
    pltpu.sync_copy(e0_hbm.at[pl.ds(w * qn, qn)], q_v)
    for k in range(qn // 16):
        qr_v[pl.ds(k * 16, 16)] = plsc.load_gather(
            fsl_v, [q_v[pl.ds(k * 16, 16)]])
    pltpu.sync_copy(qr_v, repu.at[pl.ds(w * qn, qn)])

    pltpu.sync_copy(e1_hbm.at[pl.ds(w * qn, qn)], q_v)
    for k in range(qn // 16):
        qr_v[pl.ds(k * 16, 16)] = plsc.load_gather(
            fsl_v, [q_v[pl.ds(k * 16, 16)]])
    pltpu.sync_copy(qr_v, repv.at[pl.ds(w * qn, qn)])


_sc_reps = pl.kernel(
    _sc_reps_body,
    out_type=(
        jax.ShapeDtypeStruct((NW, ETR, 128), jnp.int32),
        jax.ShapeDtypeStruct((B,), jnp.int32),
        jax.ShapeDtypeStruct((B,), jnp.int32),
    ),
    mesh=_mesh,
    compiler_params=pltpu.CompilerParams(needs_layout_passes=False),
    scratch_types=(
        pltpu.VMEM((NP,), jnp.int32),
        pltpu.VMEM((ETR, 128), jnp.int32),
        pltpu.VMEM((ETR, 128), jnp.int32),
        pltpu.VMEM((32,), jnp.int32),
        pltpu.VMEM((32,), jnp.int32),
    ),
)


def _sc_prop_body(h, row3, col3, zs, pp, row_v, col_v, rb0, rb1, zb_v,
                  acc, g0, g1, s0, s1, zsem):
    c = lax.axis_index("c")
    s = lax.axis_index("s")
    w = s * NC + c
    nz = 40

    for r in range(nz):
        for k in range(h // 16):
            zb_v[r, pl.ds(k * 16, 16)] = jnp.zeros((16,), jnp.float32)
    zd = []
    for t in range(NROWS // nz):
        zd.append(pltpu.async_copy(
            zb_v, acc.at[pl.ds(s * NROWS + t * nz, nz)], zsem))
    for d in zd:
        d.wait()
    plsc.subcore_barrier()

    pltpu.sync_copy(row3.at[w], row_v)
    pltpu.sync_copy(col3.at[w], col_v)

    nch = ETR
    rbufs = (rb0, rb1)
    gsems = (g0, g1)
    ssems = (s0, s1)
    gd = [None] * nch
    sd = [None] * nch
    gd[0] = pltpu.async_copy(zs.at[col_v.at[0]], rbufs[0], gsems[0])
    for j in range(nch):
        gd[j].wait()
        if j + 1 < nch:
            if j >= 1:
                sd[j - 1].wait()
            gd[j + 1] = pltpu.async_copy(zs.at[col_v.at[j + 1]],
                                         rbufs[(j + 1) % 2],
                                         gsems[(j + 1) % 2])
        sd[j] = pltpu.async_copy(rbufs[j % 2], acc.at[row_v.at[j]],
                                 ssems[j % 2], add=True)
    sd[nch - 2].wait()
    sd[nch - 1].wait()
    plsc.subcore_barrier()

    pltpu.sync_copy(acc.at[pl.ds(s * NROWS, NROWS)],
                    pp.at[c, pl.ds(s * NROWS, NROWS)])


def _make_sc_prop(h):
    return pl.kernel(
        functools.partial(_sc_prop_body, h),
        out_type=jax.ShapeDtypeStruct((NC, NP, h), jnp.float32),
        mesh=_mesh,
        compiler_params=pltpu.CompilerParams(needs_layout_passes=False),
        scratch_types=(
            pltpu.VMEM((ETR, 128), jnp.int32),
            pltpu.VMEM((ETR, 128), jnp.int32),
            pltpu.VMEM((128, h), jnp.float32),
            pltpu.VMEM((128, h), jnp.float32),
            pltpu.VMEM((40, h), jnp.float32),
            pltpu.VMEM_SHARED((NP, h), jnp.float32),
            pltpu.SemaphoreType.DMA,
            pltpu.SemaphoreType.DMA,
            pltpu.SemaphoreType.DMA,
            pltpu.SemaphoreType.DMA,
            pltpu.SemaphoreType.DMA,
        ),
    )


_sc_prop_hid = _make_sc_prop(HID)


def _sc_rows_body(rep2, col2, val2, rows_hbm,
                  pk_v, val_v, fic_v, valc_v, fic2_v, valc2_v,
                  zb_v, buf, sem):
    c = lax.axis_index("c")
    s = lax.axis_index("s")
    ne = ET * 2

    pltpu.sync_copy(rep2.at[s], pk_v)
    pltpu.sync_copy(col2.at[s], fic_v.at[pl.ds(0, ET * 2)])
    pltpu.sync_copy(val2.at[s], val_v)

    def prep(i, _):
        rep = pk_v[pl.ds(i * 16, 16)]
        col = fic_v[pl.ds(i * 16, 16)]
        fi0 = (rep & (CS - 1)) * ROWW + col
        tc = (rep >> 7) + 1
        pk_v[pl.ds(i * 16, 16)] = fi0 | (tc << 21)
        return 0
    lax.fori_loop(0, ne // 16, prep, 0)

    for i in range(4096 // 16):
        zb_v[pl.ds(i * 16, 16)] = jnp.zeros((16,), jnp.float32)

    words_per_tile = (CS * ROWW) // NS
    nzcopy = words_per_tile // 4096

    def chunk_body(chunk, _):
        cc = c * (NCHUNK // NC) + chunk
        base = cc * CS

        zd = []
        for z in range(nzcopy):
            zd.append(pltpu.async_copy(
                zb_v, buf.at[pl.ds(s * words_per_tile + z * 4096, 4096)],
                sem))
        for d in zd:
            d.wait()

        @pl.when(s == 0)
        def _():
            pltpu.sync_copy(zb_v.at[pl.ds(0, 8)],
                            buf.at[pl.ds(CS * ROWW, 8)])
        plsc.subcore_barrier()

        def compact(i, cnt):
            pk = pk_v[pl.ds(i * 16, 16)]
            fi = pk & ((1 << 21) - 1)
            va = val_v[pl.ds(i * 16, 16)]
            m = (pk >> 21) == cc + 1
            plsc.store_compressed(fic_v.at[pl.ds(cnt, 16)], fi, mask=m)
            plsc.store_compressed(valc_v.at[pl.ds(cnt, 16)], va, mask=m)
            pc = plsc.all_reduce_population_count(m)
            return cnt + pc[0]
        cnt = lax.fori_loop(0, ne // 16, compact, jnp.int32(0))

        for t in range(8):
            fic_v[pl.ds(cnt + t * 16, 16)] = jnp.full((16,), DUMMY,
                                                      jnp.int32)
            valc_v[pl.ds(cnt + t * 16, 16)] = jnp.zeros((16,), jnp.float32)

        for r in range(ne // 128):
            @pl.when(cnt > r * 128)
            def _():
                for t in range(8):
                    fic2_v[r % 2, pl.ds(t * 16, 16)] = \
                        fic_v[pl.ds(r * 128 + t * 16, 16)]
                    valc2_v[r % 2, pl.ds(t * 16, 16)] = \
                        valc_v[pl.ds(r * 128 + t * 16, 16)]
                pltpu.sync_copy(valc2_v.at[r % 2],
                                buf.at[fic2_v.at[r % 2]], add=True)
        plsc.subcore_barrier()

        dd = []
        for r in range(CS // NS):
            slot = s * (CS // NS) + r
            dd.append(pltpu.async_copy(
                buf.at[pl.ds(slot * ROWW, ROWW)],
                rows_hbm.at[base + slot], sem))
        for d in dd:
            d.wait()
        plsc.subcore_barrier()
        return 0

    lax.fori_loop(0, NCHUNK // NC, chunk_body, 0)


_sc_rows = pl.kernel(
    _sc_rows_body,
    out_type=jax.ShapeDtypeStruct((2 * B, ROWW), jnp.float32),
    mesh=_mesh,
    compiler_params=pltpu.CompilerParams(needs_layout_passes=False),
    scratch_types=(
        pltpu.VMEM((ET * 2,), jnp.int32),
        pltpu.VMEM((ET * 2,), jnp.float32),
        pltpu.VMEM((ET * 2 + 144,), jnp.int32),
        pltpu.VMEM((ET * 2 + 144,), jnp.float32),
        pltpu.VMEM((2, 128), jnp.int32),
        pltpu.VMEM((2, 128), jnp.float32),
        pltpu.VMEM((4096,), jnp.float32),
        pltpu.VMEM_SHARED((CS * ROWW + 8,), jnp.float32),
        pltpu.SemaphoreType.DMA,
    ),
)


def _fe_body(a_ref, w1, b1, w2, b2, o_ref):
    x = a_ref[...]
    acc = jnp.full(x.shape, b2[0], jnp.float32)
    for k in range(8):
        acc = acc + jax.nn.relu(x * w1[0, k] + b1[k]) * w2[k, 0]
    o_ref[...] = acc


def _fe_mlp(a, w1, b1, w2, b2):
    m = E // 128
    return pl.pallas_call(
        _fe_body,
        in_specs=[
            pl.BlockSpec((m, 128), lambda: (0, 0)),
            pl.BlockSpec(memory_space=pltpu.SMEM),
            pl.BlockSpec(memory_space=pltpu.SMEM),
            pl.BlockSpec(memory_space=pltpu.SMEM),
            pl.BlockSpec(memory_space=pltpu.SMEM),
        ],
        out_specs=pl.BlockSpec((m, 128), lambda: (0, 0)),
        out_shape=jax.ShapeDtypeStruct((m, 128), jnp.float32),
    )(a, w1, b1, w2, b2)


def _node_prep_body(degp, nsfp, w1, b1, w2, b2, dinv_ref, g_ref):
    deg = 1.0 + degp[0, :] + degp[1, :]
    dinv = lax.rsqrt(jnp.maximum(deg, 1e-12))
    dinv_ref[...] = dinv[:, None]
    nsf = (nsfp[0, :] + nsfp[1, :])[:, None]
    hid = jax.nn.relu(nsf * w1[...] + b1[...][None, :])
    w2r = w2[...].reshape(1, HID)
    fn = jnp.sum(hid * w2r, axis=1, keepdims=True) + b2[...][None, :]
    g_ref[...] = fn * fn


def _node_prep(degp, nsfp, w1, b1, w2, b2):
    return pl.pallas_call(
        _node_prep_body,
        out_shape=(
            jax.ShapeDtypeStruct((NP, 1), jnp.float32),
            jax.ShapeDtypeStruct((NP, 1), jnp.float32),
        ),
    )(degp, nsfp, w1, b1, w2, b2)


def _ms_body(h_ref, w_ref, dinv_ref, o_ref):
    z = jnp.dot(h_ref[...], w_ref[...], preferred_element_type=jnp.float32)
    o_ref[...] = z * dinv_ref[...]


def _matmul_scale(h, w, dinv):
    din, dout = w.shape
    bm = 1024
    return pl.pallas_call(
        _ms_body,
        grid=(NP // bm,),
        in_specs=[
            pl.BlockSpec((bm, din), lambda i: (i, 0)),
            pl.BlockSpec((din, dout), lambda i: (0, 0)),
            pl.BlockSpec((bm, 1), lambda i: (i, 0)),
        ],
        out_specs=pl.BlockSpec((bm, dout), lambda i: (i, 0)),
        out_shape=jax.ShapeDtypeStruct((NP, dout), jnp.float32),
    )(h, w, dinv)


def _ep_body(relu, pp_ref, zs_ref, dinv_ref, b_ref, o_ref):
    v = (pp_ref[0] + pp_ref[1] + zs_ref[...]) * dinv_ref[...] + b_ref[...]
    o_ref[...] = jax.nn.relu(v) if relu else v


def _epilogue(pp, zs, dinv, b, relu):
    h = zs.shape[1]
    bm = 1024
    return pl.pallas_call(
        functools.partial(_ep_body, relu),
        grid=(NP // bm,),
        in_specs=[
            pl.BlockSpec((NC, bm, h), lambda i: (0, i, 0)),
            pl.BlockSpec((bm, h), lambda i: (i, 0)),
            pl.BlockSpec((bm, 1), lambda i: (i, 0)),
            pl.BlockSpec((1, h), lambda i: (0, 0)),
        ],
        out_specs=pl.BlockSpec((bm, h), lambda i: (i, 0)),
        out_shape=jax.ShapeDtypeStruct((NP, h), jnp.float32),
    )(pp, zs, dinv, b)


def _pair_body(ru, rv, e0, e1, rowsu_ref, rowsv_ref, h3u_ref, h3v_ref,
               g_ref, osr_ref, of_ref):
    prod = rowsu_ref[0] * rowsv_ref[0] * g_ref[0]
    osr_ref[...] = jnp.full((1, 1, 128), jnp.sum(prod), jnp.float32)
    of_ref[...] = jnp.full((1, 1, 128),
                           jnp.sum(h3u_ref[...] * h3v_ref[...]), jnp.float32)


def _pair(rows3, h3, g3, ru, rv, e0, e1):
    grid_spec = pltpu.PrefetchScalarGridSpec(
        num_scalar_prefetch=4,
        grid=(B,),
        in_specs=[
            pl.BlockSpec((1, NP // 128, 128),
                         lambda b, ru, rv, e0, e1: (ru[b], 0, 0)),
            pl.BlockSpec((1, NP // 128, 128),
                         lambda b, ru, rv, e0, e1: (rv[b], 0, 0)),
            pl.BlockSpec((1, 1, 128), lambda b, ru, rv, e0, e1: (e0[b], 0, 0)),
            pl.BlockSpec((1, 1, 128), lambda b, ru, rv, e0, e1: (e1[b], 0, 0)),
            pl.BlockSpec((1, NP // 128, 128),
                         lambda b, ru, rv, e0, e1: (0, 0, 0)),
        ],
        out_specs=[
            pl.BlockSpec((1, 1, 128), lambda b, ru, rv, e0, e1: (b, 0, 0)),
            pl.BlockSpec((1, 1, 128), lambda b, ru, rv, e0, e1: (b, 0, 0)),
        ],
    )
    return pl.pallas_call(
        _pair_body,
        grid_spec=grid_spec,
        out_shape=(
            jax.ShapeDtypeStruct((B, 1, 128), jnp.float32),
            jax.ShapeDtypeStruct((B, 1, 128), jnp.float32),
        ),
    )(ru, rv, e0, e1, rows3, rows3, h3, h3, g3)


def _final_body(osr_ref, of_ref, w1, b1, w2, b2, alpha_ref,
                out_ref, os_ref):
    x = osr_ref[...]
    hid = jax.nn.relu(x * w1[...] + b1[...][None, :])
    w2r = w2[...].reshape(1, HID)
    t = jnp.sum(hid * w2r, axis=1, keepdims=True) + b2[...][None, :]
    os_v = 1.0 / (1.0 + jnp.exp(-t))
    a2 = alpha_ref[...]
    m = jnp.max(a2)
    e = jnp.exp(a2 - m)
    ssum = jnp.sum(e)
    ii = lax.broadcasted_iota(jnp.int32, (1, 2), 1)
    a0 = jnp.sum(jnp.where(ii == 0, e, 0.0)) / ssum
    a1 = jnp.sum(jnp.where(ii == 1, e, 0.0)) / ssum
    os_ref[...] = os_v
    out_ref[...] = a0 * os_v + a1 * of_ref[...] + 1e-15


def _final(osr, of, w1, b1, w2, b2, alpha):
    return pl.pallas_call(
        _final_body,
        out_shape=(
            jax.ShapeDtypeStruct((B, 1), jnp.float32),
            jax.ShapeDtypeStruct((B, 1), jnp.float32),
        ),
    )(osr, of, w1, b1, w2, b2, alpha)


def kernel(x, edge_index, edge, A_values, W0, b0, W1, b1, W2, b2,
           fe_W1, fe_b1, fe_W2, fe_b2, fn_W1, fn_b1, fn_W2, fn_b2,
           gp_W1, gp_b1, gp_W2, gp_b2, alpha):
    row, col = edge_index[0], edge_index[1]
    e0, e1 = edge[0], edge[1]

    pad = EP - E
    row_p = jnp.pad(row, (0, pad), constant_values=NP - 1)
    col_p = jnp.pad(col, (0, pad), constant_values=NP - 1)
    row3 = row_p.reshape(NW, ETR, 128)
    col3 = col_p.reshape(NW, ETR, 128)
    ones3 = jnp.pad(jnp.ones((E,), jnp.float32), (0, pad)
                    ).reshape(NW, ETR, 128)
    nodes4 = jnp.concatenate([e0, e1]).reshape(NS, 1, 128)

    ew = _fe_mlp(A_values.reshape(E // 128, 128),
                 fe_W1, fe_b1, fe_W2, fe_b2).reshape(E)
    ew3 = jnp.pad(ew, (0, pad)).reshape(NW, ETR, 128)

    degp, nsfp, fslot = _sc_stats(col3, ew3, ones3, nodes4)
    dinv, g = _node_prep(degp, nsfp, fn_W1, fn_b1, fn_W2, fn_b2)

    rep3, repu, repv = _sc_reps(fslot, row3, e0, e1)

    x_p = jnp.pad(x, ((0, NP - N), (0, 0)))
    zs0 = _matmul_scale(x_p, W0, dinv)
    pp0 = _sc_prop_hid(row3, col3, zs0)
    h1 = _epilogue(pp0, zs0, dinv, b0[None, :], True)
    zs1 = _matmul_scale(h1, W1, dinv)
    pp1 = _sc_prop_hid(row3, col3, zs1)
    h2 = _epilogue(pp1, zs1, dinv, b1[None, :], True)
    W2p = jnp.pad(W2, ((0, 0), (0, 128 - OUT)))
    b2p = jnp.pad(b2, (0, 128 - OUT))
    zs2 = _matmul_scale(h2, W2p, dinv)
    pp2 = _sc_prop_hid(row3, col3, zs2)
    h3 = _epilogue(pp2, zs2, dinv, b2p[None, :], False)

    val2 = jnp.pad(A_values, (0, pad)).reshape(NS, ET * 2)
    rep2 = rep3.reshape(NS, ET * 2)
    col2 = col_p.reshape(NS, ET * 2)
    rows = _sc_rows(rep2, col2, val2)

    rows3 = rows.reshape(2 * B, NP // 128, 128)
    g3 = g.reshape(1, NP // 128, 128)
    h3r = h3.reshape(NP, 1, 128)
    osr_f, of_f = _pair(rows3, h3r, g3, repu, repv, e0, e1)
    osr = osr_f[:, 0, :1]
    of = of_f[:, 0, :1]

    out, out_struct = _final(osr, of, gp_W1, gp_b1, gp_W2, gp_b2, alpha)
    return out, out_struct, of
